# baseline (speedup 1.0000x reference)
import jax
import jax.numpy as jnp
from jax import lax
from jax.experimental import pallas as pl
from jax.experimental.pallas import tpu as pltpu
from jax.experimental.pallas import tpu_sc as plsc

_ROWS = 4880
_DIM = 128
_TOTAL = _ROWS * _DIM
_NUM_CORES = 2
_NUM_SUBCORES = 16
_NW = _NUM_CORES * _NUM_SUBCORES
_CHUNK = _TOTAL // _NW
_NPIECE = 4
_PIECE = _CHUNK // _NPIECE


def _copy_body(src, out, buf0, buf1, rs0, rs1, ws0, ws1):
    wid = lax.axis_index("s") * _NUM_CORES + lax.axis_index("c")
    base = wid * _CHUNK
    bufs, rsems, wsems = (buf0, buf1), (rs0, rs1), (ws0, ws1)

    def rd(p):
        return pltpu.async_copy(
            src.at[pl.ds(base + p * _PIECE, _PIECE)], bufs[p % 2], rsems[p % 2])

    def wr(p):
        return pltpu.async_copy(
            bufs[p % 2], out.at[pl.ds(base + p * _PIECE, _PIECE)], wsems[p % 2])

    r0, r1 = rd(0), rd(1)
    r0.wait()
    w0 = wr(0)
    r1.wait()
    w1 = wr(1)
    w0.wait()
    r2 = rd(2)
    w1.wait()
    r3 = rd(3)
    r2.wait()
    w2 = wr(2)
    r3.wait()
    w3 = wr(3)
    w2.wait()
    w3.wait()


@jax.jit
def kernel(table):
    flat = table.reshape(_TOTAL)
    mesh = plsc.VectorSubcoreMesh(core_axis_name="c", subcore_axis_name="s")
    out = pl.kernel(
        _copy_body,
        out_type=jax.ShapeDtypeStruct((_TOTAL,), jnp.float32),
        scratch_types=[
            pltpu.VMEM((_PIECE,), jnp.float32),
            pltpu.VMEM((_PIECE,), jnp.float32),
            pltpu.SemaphoreType.DMA,
            pltpu.SemaphoreType.DMA,
            pltpu.SemaphoreType.DMA,
            pltpu.SemaphoreType.DMA,
        ],
        mesh=mesh,
    )(flat)
    return out.reshape(_ROWS, _DIM)


# speedup vs baseline: 1.0039x; 1.0039x over previous
import jax
import jax.numpy as jnp
from jax import lax
from jax.experimental import pallas as pl
from jax.experimental.pallas import tpu as pltpu
from jax.experimental.pallas import tpu_sc as plsc

_ROWS = 4880
_DIM = 128
_TOTAL = _ROWS * _DIM
_NUM_CORES = 2
_NUM_SUBCORES = 16
_NW = _NUM_CORES * _NUM_SUBCORES
_CHUNK = _TOTAL // _NW
_NPIECE = 2
_PIECE = _CHUNK // _NPIECE


def _copy_body(src, out, buf0, buf1, rs0, rs1, ws0, ws1):
    wid = lax.axis_index("s") * _NUM_CORES + lax.axis_index("c")
    base = wid * _CHUNK
    bufs, rsems, wsems = (buf0, buf1), (rs0, rs1), (ws0, ws1)

    def rd(p):
        return pltpu.async_copy(
            src.at[pl.ds(base + p * _PIECE, _PIECE)], bufs[p % 2], rsems[p % 2])

    def wr(p):
        return pltpu.async_copy(
            bufs[p % 2], out.at[pl.ds(base + p * _PIECE, _PIECE)], wsems[p % 2])

    r0 = rd(0)
    r0.wait()
    w0 = wr(0)
    r1 = rd(1)
    r1.wait()
    w1 = wr(1)
    w0.wait()
    w1.wait()


@jax.jit
def kernel(table):
    flat = table.reshape(_TOTAL)
    mesh = plsc.VectorSubcoreMesh(core_axis_name="c", subcore_axis_name="s")
    out = pl.kernel(
        _copy_body,
        out_type=jax.ShapeDtypeStruct((_TOTAL,), jnp.float32),
        scratch_types=[
            pltpu.VMEM((_PIECE,), jnp.float32),
            pltpu.VMEM((_PIECE,), jnp.float32),
            pltpu.SemaphoreType.DMA,
            pltpu.SemaphoreType.DMA,
            pltpu.SemaphoreType.DMA,
            pltpu.SemaphoreType.DMA,
        ],
        mesh=mesh,
    )(flat)
    return out.reshape(_ROWS, _DIM)


# speedup vs baseline: 1.0325x; 1.0286x over previous
import functools

import jax
import jax.numpy as jnp
from jax import lax
from jax.experimental import pallas as pl
from jax.experimental.pallas import tpu as pltpu
from jax.experimental.pallas import tpu_sc as plsc

_ROWS = 4880
_DIM = 128
_TOTAL = _ROWS * _DIM
_NUM_CORES = 2
_NUM_SUBCORES = 16
_NW = _NUM_CORES * _NUM_SUBCORES
_CHUNK = _TOTAL // _NW


def _copy_body(src_hbm, out_hbm, buf):
    wid = lax.axis_index("s") * _NUM_CORES + lax.axis_index("c")
    base = wid * _CHUNK
    pltpu.sync_copy(src_hbm.at[pl.ds(base, _CHUNK)], buf)
    pltpu.sync_copy(buf, out_hbm.at[pl.ds(base, _CHUNK)])


@jax.jit
def kernel(table):
    flat = table.reshape(_TOTAL)
    mesh = plsc.VectorSubcoreMesh(core_axis_name="c", subcore_axis_name="s")
    out = pl.kernel(
        _copy_body,
        out_type=jax.ShapeDtypeStruct((_TOTAL,), jnp.float32),
        scratch_types=[pltpu.VMEM((_CHUNK,), jnp.float32)],
        mesh=mesh,
    )(flat)
    return out.reshape(_ROWS, _DIM)
